# Initial kernel scaffold; baseline (speedup 1.0000x reference)
import jax
import jax.numpy as jnp
from jax.experimental import pallas as pl


def kernel(x, embedding):
    raise NotImplementedError("write your pallas kernel here")


# speedup vs baseline: 4.3989x; 4.3989x over previous
import functools

import jax
import jax.numpy as jnp
from jax import lax
from jax.experimental import pallas as pl
from jax.experimental.pallas import tpu as pltpu
from jax.experimental.pallas import tpu_sc as plsc

L = 16
NC, NS = 2, 16
NW = NC * NS


def _make_sc_lookup(BC, M, P):
  CHUNK = P // NW
  GROUPS = CHUNK // L
  mesh = plsc.VectorSubcoreMesh(core_axis_name="c", subcore_axis_name="s")

  @functools.partial(
      pl.kernel,
      out_type=jax.ShapeDtypeStruct((BC, M, P), jnp.float32),
      mesh=mesh,
      scratch_types=[
          pltpu.VMEM((256 * M,), jnp.float32),
          pltpu.VMEM((CHUNK,), jnp.float32),
          pltpu.VMEM((M, CHUNK), jnp.float32),
      ],
      compiler_params=pltpu.CompilerParams(
          use_tc_tiling_on_sc=False, needs_layout_passes=False),
  )
  def lookup(emb_hbm, x_hbm, out_hbm, tab_v, xin_v, outb_v):
    wid = lax.axis_index("s") * NC + lax.axis_index("c")
    base = wid * CHUNK

    pltpu.sync_copy(emb_hbm, tab_v)

    def binarize(g, _):
      v = tab_v[pl.ds(g * L, L)]
      tab_v[pl.ds(g * L, L)] = (jnp.sign(v) + 1.0) * 0.5
      return _

    lax.fori_loop(0, (256 * M) // L, binarize, None)

    def plane(i, _):
      pltpu.sync_copy(x_hbm.at[i, pl.ds(base, CHUNK)], xin_v)

      def group(g, _):
        xv = xin_v[pl.ds(g * L, L)]
        tbase = (xv * 255.0).astype(jnp.int32) * M
        for mm in range(M):
          val = plsc.load_gather(tab_v, [tbase + mm])
          outb_v[mm, pl.ds(g * L, L)] = val
        return _

      lax.fori_loop(0, GROUPS, group, None)
      pltpu.sync_copy(outb_v, out_hbm.at[i, :, pl.ds(base, CHUNK)])
      return _

    lax.fori_loop(0, BC, plane, None)

  return lookup


def kernel(x, embedding):
  b, c, h, w = x.shape
  m = embedding.shape[1]
  P = h * w
  BC = b * c
  xf = x.reshape(BC, P)
  emb_flat = embedding.reshape(256 * m)
  out = _make_sc_lookup(BC, m, P)(emb_flat, xf)
  return out.reshape(b, c * m, h, w)


# speedup vs baseline: 14.0193x; 3.1870x over previous
import functools

import jax
import jax.numpy as jnp
from jax import lax
from jax.experimental import pallas as pl
from jax.experimental.pallas import tpu as pltpu
from jax.experimental.pallas import tpu_sc as plsc

L = 16
NC, NS = 2, 16
NW = NC * NS


def _make_sc_lookup(BC, M, P):
  CHUNK = P // NW
  GROUPS = CHUNK // L
  mesh = plsc.VectorSubcoreMesh(core_axis_name="c", subcore_axis_name="s")

  @functools.partial(
      pl.kernel,
      out_type=jax.ShapeDtypeStruct((BC, M, P), jnp.float32),
      mesh=mesh,
      scratch_types=[
          pltpu.VMEM((256 * M,), jnp.float32),
          pltpu.VMEM((256 * M,), jnp.float32),
          pltpu.VMEM((CHUNK,), jnp.float32),
          pltpu.VMEM((M, CHUNK), jnp.float32),
          pltpu.VMEM((M, CHUNK), jnp.float32),
          pltpu.SemaphoreType.DMA,
          pltpu.SemaphoreType.DMA,
      ],
      compiler_params=pltpu.CompilerParams(
          use_tc_tiling_on_sc=False, needs_layout_passes=False),
  )
  def lookup(emb_hbm, x_hbm, out_hbm, ntab_v, tab_v, xin_v, outb0, outb1,
             sem0, sem1):
    wid = lax.axis_index("s") * NC + lax.axis_index("c")
    base = wid * CHUNK

    pltpu.sync_copy(emb_hbm, ntab_v)
    kiota = lax.iota(jnp.int32, L)

    def transpose_m(m, _):
      def transpose_k(kg, _):
        kv = (kiota + kg * L) * M + m
        v = plsc.load_gather(ntab_v, [kv])
        tab_v[pl.ds(m * 256 + kg * L, L)] = (jnp.sign(v) + 1.0) * 0.5
        return _

      lax.fori_loop(0, 256 // L, transpose_k, None)
      return _

    lax.fori_loop(0, M, transpose_m, None)

    def do_loads(g):
      xv = xin_v[pl.ds(g * L, L)]
      tb = (xv * 255.0).astype(jnp.int32)
      return tuple(plsc.load_gather(tab_v, [tb + mm * 256]) for mm in range(M))

    def do_stores(g, vals, outb):
      for mm in range(M):
        outb[mm, pl.ds(g * L, L)] = vals[mm]

    def compute_plane(outb):
      vals0 = do_loads(0)

      def group(g, vals):
        nvals = do_loads(g)
        do_stores(g - 1, vals, outb)
        return nvals

      last = lax.fori_loop(1, GROUPS, group, vals0)
      do_stores(GROUPS - 1, last, outb)

    def out_copy(i, outb, sem):
      return pltpu.make_async_copy(
          outb, out_hbm.at[i, :, pl.ds(base, CHUNK)], sem)

    def pair(k, _):
      i = 2 * k
      pltpu.sync_copy(x_hbm.at[i, pl.ds(base, CHUNK)], xin_v)

      @pl.when(k > 0)
      def _wait0():
        out_copy(i - 2, outb0, sem0).wait()

      compute_plane(outb0)
      out_copy(i, outb0, sem0).start()

      pltpu.sync_copy(x_hbm.at[i + 1, pl.ds(base, CHUNK)], xin_v)

      @pl.when(k > 0)
      def _wait1():
        out_copy(i - 1, outb1, sem1).wait()

      compute_plane(outb1)
      out_copy(i + 1, outb1, sem1).start()
      return _

    lax.fori_loop(0, BC // 2, pair, None)
    out_copy(BC - 2, outb0, sem0).wait()
    out_copy(BC - 1, outb1, sem1).wait()

  return lookup


def kernel(x, embedding):
  b, c, h, w = x.shape
  m = embedding.shape[1]
  P = h * w
  BC = b * c
  xf = x.reshape(BC, P)
  emb_flat = embedding.reshape(256 * m)
  out = _make_sc_lookup(BC, m, P)(emb_flat, xf)
  return out.reshape(b, c * m, h, w)


# speedup vs baseline: 24.3806x; 1.7391x over previous
import functools

import jax
import jax.numpy as jnp
from jax import lax
from jax.experimental import pallas as pl
from jax.experimental.pallas import tpu as pltpu
from jax.experimental.pallas import tpu_sc as plsc

L = 16
NC, NS = 2, 16
NW = NC * NS
RB = 8


def _make_sc_lookup(B, C, M, H, W):
  HB = H // RB
  NTASK = B * C * HB
  TPW = NTASK // NW
  MH = M // 2
  GPR = W // L
  mesh = plsc.VectorSubcoreMesh(core_axis_name="c", subcore_axis_name="s")

  @functools.partial(
      pl.kernel,
      out_type=jax.ShapeDtypeStruct((B, C * M, H, W), jnp.float32),
      mesh=mesh,
      scratch_types=[
          pltpu.VMEM((256 * M,), jnp.float32),
          pltpu.VMEM((256 * M,), jnp.float32),
          pltpu.VMEM((RB, W), jnp.float32),
          pltpu.VMEM((MH, RB, W), jnp.float32),
          pltpu.VMEM((MH, RB, W), jnp.float32),
          pltpu.SemaphoreType.DMA,
          pltpu.SemaphoreType.DMA,
      ],
      compiler_params=pltpu.CompilerParams(
          use_tc_tiling_on_sc=True, needs_layout_passes=False),
  )
  def lookup(emb_hbm, x_hbm, out_hbm, ntab_v, tab_v, xin_v, outb0, outb1,
             sem0, sem1):
    wid = lax.axis_index("s") * NC + lax.axis_index("c")

    pltpu.sync_copy(emb_hbm, ntab_v)
    kiota = lax.iota(jnp.int32, L)

    def transpose_m(m, _):
      def transpose_k(kg, _):
        kv = (kiota + kg * L) * M + m
        v = plsc.load_gather(ntab_v, [kv])
        tab_v[pl.ds(m * 256 + kg * L, L)] = (jnp.sign(v) + 1.0) * 0.5
        return _

      lax.fori_loop(0, 256 // L, transpose_k, None)
      return _

    lax.fori_loop(0, M, transpose_m, None)

    def do_loads(r, cg, mbase):
      xv = xin_v[r, pl.ds(cg * L, L)]
      tb = (xv * 255.0).astype(jnp.int32)
      return tuple(
          plsc.load_gather(tab_v, [tb + (mbase + mm) * 256])
          for mm in range(MH))

    def do_stores(r, cg, vals, outb):
      for mm in range(MH):
        outb[mm, r, pl.ds(cg * L, L)] = vals[mm]

    def compute_half(mbase, outb):
      vals0 = do_loads(0, 0, mbase)

      def group(g, vals):
        r, cg = g // GPR, g % GPR
        nvals = do_loads(r, cg, mbase)
        pr, pcg = (g - 1) // GPR, (g - 1) % GPR
        do_stores(pr, pcg, vals, outb)
        return nvals

      last = lax.fori_loop(1, RB * GPR, group, vals0)
      do_stores(RB - 1, GPR - 1, last, outb)

    def out_copy(b, c, rb, mbase, outb, sem):
      return pltpu.make_async_copy(
          outb,
          out_hbm.at[b, pl.ds(c * M + mbase, MH),
                     pl.ds(pl.multiple_of(rb * RB, RB), RB), :],
          sem)

    def task(t, _):
      g = wid * TPW + t
      plane = g // HB
      rb = g % HB
      b = plane // C
      c = plane % C
      pltpu.sync_copy(
          x_hbm.at[b, c, pl.ds(pl.multiple_of(rb * RB, RB), RB), :], xin_v)

      @pl.when(t > 0)
      def _w0():
        out_copy(0, 0, 0, 0, outb0, sem0).wait()

      compute_half(0, outb0)
      out_copy(b, c, rb, 0, outb0, sem0).start()

      @pl.when(t > 0)
      def _w1():
        out_copy(0, 0, 0, 0, outb1, sem1).wait()

      compute_half(MH, outb1)
      out_copy(b, c, rb, MH, outb1, sem1).start()
      return _

    lax.fori_loop(0, TPW, task, None)
    out_copy(0, 0, 0, 0, outb0, sem0).wait()
    out_copy(0, 0, 0, 0, outb1, sem1).wait()

  return lookup


def kernel(x, embedding):
  b, c, h, w = x.shape
  m = embedding.shape[1]
  emb_flat = embedding.reshape(256 * m)
  return _make_sc_lookup(b, c, m, h, w)(emb_flat, x)


# speedup vs baseline: 31.0712x; 1.2744x over previous
import functools

import jax
import jax.numpy as jnp
from jax import lax
from jax.experimental import pallas as pl
from jax.experimental.pallas import tpu as pltpu
from jax.experimental.pallas import tpu_sc as plsc

L = 16
NC, NS = 2, 16
NW = NC * NS
RB = 8


def _make_sc_lookup(B, C, M, H, W):
  HB = H // RB
  NTASK = B * C * HB
  TPW = NTASK // NW
  MH = M // 2
  PH = MH // 2
  MP = M // 2
  GPR = W // L
  mesh = plsc.VectorSubcoreMesh(core_axis_name="c", subcore_axis_name="s")

  @functools.partial(
      pl.kernel,
      out_type=jax.ShapeDtypeStruct((B, C * M, H, W), jnp.float32),
      mesh=mesh,
      scratch_types=[
          pltpu.VMEM((256 * M,), jnp.float32),
          pltpu.VMEM((256 * MP,), jnp.int32),
          pltpu.VMEM((RB, W), jnp.float32),
          pltpu.VMEM((MH, RB, W), jnp.float32),
          pltpu.VMEM((MH, RB, W), jnp.float32),
          pltpu.SemaphoreType.DMA,
          pltpu.SemaphoreType.DMA,
      ],
      compiler_params=pltpu.CompilerParams(
          use_tc_tiling_on_sc=True, needs_layout_passes=False),
  )
  def lookup(emb_hbm, x_hbm, out_hbm, ntab_v, tab_v, xin_v, outb0, outb1,
             sem0, sem1):
    wid = lax.axis_index("s") * NC + lax.axis_index("c")

    pltpu.sync_copy(emb_hbm, ntab_v)
    kiota = lax.iota(jnp.int32, L)

    def transpose_mp(mp, _):
      def transpose_k(kg, _):
        kv = (kiota + kg * L) * M + 2 * mp
        va = plsc.load_gather(ntab_v, [kv])
        vb = plsc.load_gather(ntab_v, [kv + 1])
        ea = (jnp.sign(va) + 1.0) * 0.5
        eb = (jnp.sign(vb) + 1.0) * 0.5
        packed = plsc.pack(ea, eb, format=plsc.PackFormat.INTERLEAVED)
        tab_v[pl.ds(mp * 256 + kg * L, L)] = plsc.bitcast(packed, jnp.int32)
        return _

      lax.fori_loop(0, 256 // L, transpose_k, None)
      return _

    lax.fori_loop(0, MP, transpose_mp, None)

    def do_loads(r, cg, pbase):
      xv = xin_v[r, pl.ds(cg * L, L)]
      tb = (xv * 255.0).astype(jnp.int32)
      return tuple(
          plsc.load_gather(tab_v, [tb + (pbase + j) * 256])
          for j in range(PH))

    def do_stores(r, cg, packs, outb):
      for j in range(PH):
        a, b = plsc.unpack(
            plsc.bitcast(packs[j], jnp.bfloat16),
            format=plsc.PackFormat.INTERLEAVED)
        outb[2 * j, r, pl.ds(cg * L, L)] = a.astype(jnp.float32)
        outb[2 * j + 1, r, pl.ds(cg * L, L)] = b.astype(jnp.float32)

    def compute_half(pbase, outb):
      packs0 = do_loads(0, 0, pbase)

      def group(g, packs):
        r, cg = g // GPR, g % GPR
        npacks = do_loads(r, cg, pbase)
        pr, pcg = (g - 1) // GPR, (g - 1) % GPR
        do_stores(pr, pcg, packs, outb)
        return npacks

      last = lax.fori_loop(1, RB * GPR, group, packs0)
      do_stores(RB - 1, GPR - 1, last, outb)

    def out_copy(b, c, rb, mbase, outb, sem):
      return pltpu.make_async_copy(
          outb,
          out_hbm.at[b, pl.ds(c * M + mbase, MH),
                     pl.ds(pl.multiple_of(rb * RB, RB), RB), :],
          sem)

    def task(t, _):
      g = wid * TPW + t
      plane = g // HB
      rb = g % HB
      b = plane // C
      c = plane % C
      pltpu.sync_copy(
          x_hbm.at[b, c, pl.ds(pl.multiple_of(rb * RB, RB), RB), :], xin_v)

      @pl.when(t > 0)
      def _w0():
        out_copy(0, 0, 0, 0, outb0, sem0).wait()

      compute_half(0, outb0)
      out_copy(b, c, rb, 0, outb0, sem0).start()

      @pl.when(t > 0)
      def _w1():
        out_copy(0, 0, 0, 0, outb1, sem1).wait()

      compute_half(PH, outb1)
      out_copy(b, c, rb, MH, outb1, sem1).start()
      return _

    lax.fori_loop(0, TPW, task, None)
    out_copy(0, 0, 0, 0, outb0, sem0).wait()
    out_copy(0, 0, 0, 0, outb1, sem1).wait()

  return lookup


def kernel(x, embedding):
  b, c, h, w = x.shape
  m = embedding.shape[1]
  emb_flat = embedding.reshape(256 * m)
  return _make_sc_lookup(b, c, m, h, w)(emb_flat, x)


# speedup vs baseline: 37.8911x; 1.2195x over previous
import functools

import jax
import jax.numpy as jnp
from jax import lax
from jax.experimental import pallas as pl
from jax.experimental.pallas import tpu as pltpu
from jax.experimental.pallas import tpu_sc as plsc

L = 16
NC, NS = 2, 16
NW = NC * NS
RB = 8


def _make_sc_lookup(B, C, M, H, W):
  HB = H // RB
  NTASK = B * C * HB
  TPW = NTASK // NW
  MH = M // 2
  QH = MH // 4
  MQ = M // 4
  GPR = W // L
  mesh = plsc.VectorSubcoreMesh(core_axis_name="c", subcore_axis_name="s")

  @functools.partial(
      pl.kernel,
      out_type=jax.ShapeDtypeStruct((B, C * M, H, W), jnp.float32),
      mesh=mesh,
      scratch_types=[
          pltpu.VMEM((256 * M,), jnp.float32),
          pltpu.VMEM((256 * MQ,), jnp.int32),
          pltpu.VMEM((RB, W), jnp.float32),
          pltpu.VMEM((MH, RB, W), jnp.float32),
          pltpu.VMEM((MH, RB, W), jnp.float32),
          pltpu.SemaphoreType.DMA,
          pltpu.SemaphoreType.DMA,
      ],
      compiler_params=pltpu.CompilerParams(
          use_tc_tiling_on_sc=True, needs_layout_passes=False),
  )
  def lookup(emb_hbm, x_hbm, out_hbm, ntab_v, tab_v, xin_v, outb0, outb1,
             sem0, sem1):
    wid = lax.axis_index("s") * NC + lax.axis_index("c")

    pltpu.sync_copy(emb_hbm, ntab_v)
    kiota = lax.iota(jnp.int32, L)

    def binarize(v):
      return (jnp.sign(v) + 1.0) * 0.5

    def transpose_mq(mq, _):
      def transpose_k(kg, _):
        kv = (kiota + kg * L) * M + 4 * mq
        ea = binarize(plsc.load_gather(ntab_v, [kv]))
        eb = binarize(plsc.load_gather(ntab_v, [kv + 1]))
        ec = binarize(plsc.load_gather(ntab_v, [kv + 2]))
        ed = binarize(plsc.load_gather(ntab_v, [kv + 3]))
        pab = plsc.pack(ea, eb, format=plsc.PackFormat.INTERLEAVED)
        pcd = plsc.pack(ec, ed, format=plsc.PackFormat.INTERLEAVED)
        quad = plsc.pack(pab, pcd, format=plsc.PackFormat.INTERLEAVED,
                         preferred_element_type=jnp.float8_e4m3fn)
        tab_v[pl.ds(mq * 256 + kg * L, L)] = plsc.bitcast(quad, jnp.int32)
        return _

      lax.fori_loop(0, 256 // L, transpose_k, None)
      return _

    lax.fori_loop(0, MQ, transpose_mq, None)

    def do_loads(r, cg, qbase):
      xv = xin_v[r, pl.ds(cg * L, L)]
      tb = (xv * 255.0).astype(jnp.int32)
      return tuple(
          plsc.load_gather(tab_v, [tb + (qbase + j) * 256])
          for j in range(QH))

    def do_stores(r, cg, packs, outb):
      for j in range(QH):
        pab, pcd = plsc.unpack(
            plsc.bitcast(packs[j], jnp.float8_e4m3fn),
            format=plsc.PackFormat.INTERLEAVED,
            preferred_element_type=jnp.bfloat16)
        a, b = plsc.unpack(pab, format=plsc.PackFormat.INTERLEAVED)
        c, d = plsc.unpack(pcd, format=plsc.PackFormat.INTERLEAVED)
        outb[4 * j, r, pl.ds(cg * L, L)] = a
        outb[4 * j + 1, r, pl.ds(cg * L, L)] = b
        outb[4 * j + 2, r, pl.ds(cg * L, L)] = c
        outb[4 * j + 3, r, pl.ds(cg * L, L)] = d

    def compute_half(qbase, outb):
      packs0 = do_loads(0, 0, qbase)

      def group(g, packs):
        r, cg = g // GPR, g % GPR
        npacks = do_loads(r, cg, qbase)
        pr, pcg = (g - 1) // GPR, (g - 1) % GPR
        do_stores(pr, pcg, packs, outb)
        return npacks

      last = lax.fori_loop(1, RB * GPR, group, packs0)
      do_stores(RB - 1, GPR - 1, last, outb)

    def out_copy(b, c, rb, mbase, outb, sem):
      return pltpu.make_async_copy(
          outb,
          out_hbm.at[b, pl.ds(c * M + mbase, MH),
                     pl.ds(pl.multiple_of(rb * RB, RB), RB), :],
          sem)

    def task(t, _):
      g = wid * TPW + t
      plane = g // HB
      rb = g % HB
      b = plane // C
      c = plane % C
      pltpu.sync_copy(
          x_hbm.at[b, c, pl.ds(pl.multiple_of(rb * RB, RB), RB), :], xin_v)

      @pl.when(t > 0)
      def _w0():
        out_copy(0, 0, 0, 0, outb0, sem0).wait()

      compute_half(0, outb0)
      out_copy(b, c, rb, 0, outb0, sem0).start()

      @pl.when(t > 0)
      def _w1():
        out_copy(0, 0, 0, 0, outb1, sem1).wait()

      compute_half(QH, outb1)
      out_copy(b, c, rb, MH, outb1, sem1).start()
      return _

    lax.fori_loop(0, TPW, task, None)
    out_copy(0, 0, 0, 0, outb0, sem0).wait()
    out_copy(0, 0, 0, 0, outb1, sem1).wait()

  return lookup


def kernel(x, embedding):
  b, c, h, w = x.shape
  m = embedding.shape[1]
  emb_flat = embedding.reshape(256 * m)
  return _make_sc_lookup(b, c, m, h, w)(emb_flat, x)
